# Initial kernel scaffold; baseline (speedup 1.0000x reference)
import jax
import jax.numpy as jnp
from jax.experimental import pallas as pl


def kernel(x, word_emb, pos_emb):
    raise NotImplementedError("write your pallas kernel here")


# speedup vs baseline: 3.7817x; 3.7817x over previous
import functools

import jax
import jax.numpy as jnp
from jax import lax
from jax.experimental import pallas as pl
from jax.experimental.pallas import tpu as pltpu
from jax.experimental.pallas import tpu_sc as plsc

B = 1024
S = 200
D = 128
L = 16
NC = 2
NS = 16
NW = NC * NS
SEQ_PER_W = B // NW
IDX_CHUNKS = 2
IDX_CHUNK = S // IDX_CHUNKS


def _emb_body(x_hbm, word_hbm, pos_hbm, out_hbm, idx_v, rows_v, pos_v, sem):
    wid = lax.axis_index("s") * NC + lax.axis_index("c")

    pltpu.sync_copy(pos_hbm.at[pl.ds(0, S)], pos_v)

    def seq_body(i, carry):
        seq = wid * SEQ_PER_W + i
        pltpu.sync_copy(x_hbm.at[seq], idx_v)
        for j in range(IDX_CHUNKS):
            pltpu.async_copy(
                word_hbm.at[idx_v.at[j]],
                rows_v.at[pl.ds(j * IDX_CHUNK, IDX_CHUNK)],
                sem,
            ).wait()
        def add_row(r, carry2):
            for d in range(D // L):
                sl = pl.ds(d * L, L)
                rows_v[r, sl] = rows_v[r, sl] + pos_v[r, sl]
            return carry2

        lax.fori_loop(0, S, add_row, 0, unroll=2)
        pltpu.sync_copy(rows_v, out_hbm.at[seq])
        return carry

    lax.fori_loop(0, SEQ_PER_W, seq_body, 0)


@functools.cache
def _make_emb_kernel():
    return pl.kernel(
        _emb_body,
        out_type=jax.ShapeDtypeStruct((B, S, D), jnp.float32),
        mesh=plsc.VectorSubcoreMesh(
            core_axis_name="c", subcore_axis_name="s", num_cores=NC, num_subcores=NS
        ),
        scratch_types=[
            pltpu.VMEM((IDX_CHUNKS, IDX_CHUNK), jnp.int32),
            pltpu.VMEM((S, D), jnp.float32),
            pltpu.VMEM((S, D), jnp.float32),
            pltpu.SemaphoreType.DMA,
        ],
    )


@jax.jit
def kernel(x, word_emb, pos_emb):
    x = x.reshape(B, IDX_CHUNKS, IDX_CHUNK).astype(jnp.int32)
    return _make_emb_kernel()(x, word_emb, pos_emb)


# speedup vs baseline: 5.5665x; 1.4720x over previous
import functools

import jax
import jax.numpy as jnp
from jax import lax
from jax.experimental import pallas as pl
from jax.experimental.pallas import tpu as pltpu
from jax.experimental.pallas import tpu_sc as plsc

B = 1024
S = 200
D = 128
L = 16
NC = 2
NS = 16
NW = NC * NS
SEQ_PER_W = B // NW
IDX_CHUNKS = 2
IDX_CHUNK = S // IDX_CHUNKS


def _emb_body(x_hbm, word_hbm, pos_hbm, out_hbm, idx_v, rows_v, pos_v, sem):
    wid = lax.axis_index("s") * NC + lax.axis_index("c")

    pltpu.sync_copy(pos_hbm.at[pl.ds(0, S)], pos_v)

    def seq_body(i, carry):
        seq = wid * SEQ_PER_W + i
        pltpu.sync_copy(x_hbm.at[seq], idx_v)
        pltpu.sync_copy(pos_hbm.at[pl.ds(0, S)], rows_v)
        for j in range(IDX_CHUNKS):
            pltpu.async_copy(
                word_hbm.at[idx_v.at[j]],
                rows_v.at[pl.ds(j * IDX_CHUNK, IDX_CHUNK)],
                sem,
                add=True,
            ).wait()
        pltpu.sync_copy(rows_v, out_hbm.at[seq])
        return carry

    lax.fori_loop(0, SEQ_PER_W, seq_body, 0)


@functools.cache
def _make_emb_kernel():
    return pl.kernel(
        _emb_body,
        out_type=jax.ShapeDtypeStruct((B, S, D), jnp.float32),
        mesh=plsc.VectorSubcoreMesh(
            core_axis_name="c", subcore_axis_name="s", num_cores=NC, num_subcores=NS
        ),
        scratch_types=[
            pltpu.VMEM((IDX_CHUNKS, IDX_CHUNK), jnp.int32),
            pltpu.VMEM((S, D), jnp.float32),
            pltpu.VMEM((S, D), jnp.float32),
            pltpu.SemaphoreType.DMA,
        ],
    )


@jax.jit
def kernel(x, word_emb, pos_emb):
    x = x.reshape(B, IDX_CHUNKS, IDX_CHUNK).astype(jnp.int32)
    return _make_emb_kernel()(x, word_emb, pos_emb)
